# baseline (speedup 1.0000x reference)
import functools

import jax
import jax.numpy as jnp
from jax import lax
from jax.experimental import pallas as pl
from jax.experimental.pallas import tpu as pltpu
from jax.experimental.pallas import tpu_sc as plsc

NUM_E = 8
TOPK = 2
ALPHA = 0.01
T = 512
NW = 32
GCH = 64


def _router_body(x_ref, wr_ref, br_ref, gating_ref, route_ref, aux_ref):
    x = x_ref[...]
    logits = jnp.dot(x, wr_ref[...], preferred_element_type=jnp.float32)
    logits = logits + br_ref[...]
    n = logits.shape[0]
    eidx = lax.broadcasted_iota(jnp.int32, (n, NUM_E), 1)
    m0 = jnp.max(logits, axis=1, keepdims=True)
    e0 = jnp.min(jnp.where(logits == m0, eidx, NUM_E), axis=1, keepdims=True)
    masked = jnp.where(eidx == e0, -jnp.inf, logits)
    m1 = jnp.max(masked, axis=1, keepdims=True)
    e1 = jnp.min(jnp.where(masked == m1, eidx, NUM_E), axis=1, keepdims=True)
    z = jnp.exp(m1 - m0)
    w0 = 1.0 / (1.0 + z)
    w1 = z / (1.0 + z)
    gating = jnp.where(eidx == e0, w0, 0.0) + jnp.where(eidx == e1, w1, 0.0)
    gating_ref[...] = gating
    route_ref[...] = (jnp.where(eidx == 0, e0.astype(jnp.float32), 0.0)
                      + jnp.where(eidx == 1, e1.astype(jnp.float32), 0.0)
                      + jnp.where(eidx == 2, w0, 0.0)
                      + jnp.where(eidx == 3, w1, 0.0))
    imp = jnp.sum(gating, axis=0)
    mean_imp = jnp.mean(imp)
    var = jnp.sum((imp - mean_imp) ** 2) / (NUM_E - 1)
    aux_ref[...] = (ALPHA * var / (mean_imp * mean_imp + 1e-08)).reshape(1, 1)


def _run_router(x2d, Wr, br):
    n = x2d.shape[0]
    return pl.pallas_call(
        _router_body,
        out_shape=[
            jax.ShapeDtypeStruct((n, NUM_E), jnp.float32),
            jax.ShapeDtypeStruct((n, NUM_E), jnp.float32),
            jax.ShapeDtypeStruct((1, 1), jnp.float32),
        ],
    )(x2d, Wr, br.reshape(1, NUM_E))


NI = 2


def _expert_body(be_ref, fl_ref, x_ref, w1_ref, b1_ref, w2_ref, b2_ref,
                 y_ref):
    j = pl.program_id(0)
    i = pl.program_id(1)

    @pl.when(fl_ref[j] == 1)
    def _():
        x = x_ref[...]
        h = jnp.dot(x, w1_ref[0], preferred_element_type=jnp.float32,
                    precision=lax.Precision.DEFAULT)
        h = h + b1_ref[0]
        h = 0.5 * h * (1.0 + lax.erf(h * 0.7071067811865476))
        y = jnp.dot(h, w2_ref[0], preferred_element_type=jnp.float32,
                    precision=lax.Precision.DEFAULT)

        @pl.when(i == 0)
        def _():
            y_ref[...] = y + b2_ref[0]

        @pl.when(i > 0)
        def _():
            y_ref[...] = y_ref[...] + y


def _run_experts(x_sorted, W1, b1, W2, b2, block_e, flags):
    nbt, h = x_sorted.shape
    nb = nbt // T
    i_dim = W1.shape[2]
    it = i_dim // NI
    grid_spec = pltpu.PrefetchScalarGridSpec(
        num_scalar_prefetch=2,
        grid=(nb, NI),
        in_specs=[
            pl.BlockSpec((T, h), lambda j, i, be, fl: (j, 0)),
            pl.BlockSpec((1, h, it), lambda j, i, be, fl: (be[j], 0, i)),
            pl.BlockSpec((1, 1, it), lambda j, i, be, fl: (be[j], 0, i)),
            pl.BlockSpec((1, it, h), lambda j, i, be, fl: (be[j], i, 0)),
            pl.BlockSpec((1, 1, h), lambda j, i, be, fl: (be[j], 0, 0)),
        ],
        out_specs=pl.BlockSpec((T, h), lambda j, i, be, fl: (j, 0)),
    )
    return pl.pallas_call(
        _expert_body,
        grid_spec=grid_spec,
        out_shape=jax.ShapeDtypeStruct((nbt, h), jnp.float32),
    )(block_e, flags, x_sorted, W1, b1.reshape(NUM_E, 1, i_dim), W2,
      b2.reshape(NUM_E, 1, h))


def _make_dispatch(n, nbt, d):
    npw = TOPK * n // NW
    nit = npw // GCH
    mesh = plsc.VectorSubcoreMesh(core_axis_name="c", subcore_axis_name="s")

    @functools.partial(
        pl.kernel, mesh=mesh,
        out_type=jax.ShapeDtypeStruct((nbt, d), jnp.float32),
        scratch_types=[
            pltpu.VMEM((nit, GCH), jnp.int32),
            pltpu.VMEM((2, GCH, d), jnp.float32),
            pltpu.SemaphoreType.DMA,
            pltpu.SemaphoreType.DMA,
            pltpu.SemaphoreType.DMA,
            pltpu.SemaphoreType.DMA,
        ],
    )
    def dk(x_hbm, dst3_hbm, out_hbm, idx_v, rows_v, g0, g1, o0, o1):
        wid = lax.axis_index("s") * 2 + lax.axis_index("c")
        src0 = (wid % (NW // TOPK)) * npw
        pltpu.sync_copy(dst3_hbm.at[wid], idx_v)
        gsem = (g0, g1)
        osem = (o0, o1)

        def load(c):
            return pltpu.async_copy(
                x_hbm.at[pl.ds(src0 + c * GCH, GCH)],
                rows_v.at[c % 2], gsem[c % 2])

        def scat(c):
            return pltpu.async_copy(
                rows_v.at[c % 2], out_hbm.at[idx_v.at[c]], osem[c % 2])

        gd = {0: load(0)}
        od = {}
        for c in range(nit):
            if c + 1 < nit:
                if c - 1 >= 0:
                    od[c - 1].wait()
                gd[c + 1] = load(c + 1)
            gd[c].wait()
            od[c] = scat(c)
        od[nit - 2].wait()
        od[nit - 1].wait()

    return dk


CCH = 32


def _make_combine(n, d):
    npw = n // NW
    nit = npw // CCH
    mesh = plsc.VectorSubcoreMesh(core_axis_name="c", subcore_axis_name="s")

    @functools.partial(
        pl.kernel, mesh=mesh,
        out_type=jax.ShapeDtypeStruct((n, d), jnp.float32),
        scratch_types=[
            pltpu.VMEM((npw,), jnp.int32),
            pltpu.VMEM((npw,), jnp.int32),
            pltpu.VMEM((2, CCH, 16), jnp.float32),
            pltpu.VMEM((2, CCH, 16), jnp.float32),
            pltpu.VMEM((2, CCH, d), jnp.float32),
            pltpu.VMEM((2, CCH, d), jnp.float32),
            pltpu.SemaphoreType.DMA,
            pltpu.SemaphoreType.DMA,
            pltpu.SemaphoreType.DMA,
            pltpu.SemaphoreType.DMA,
            pltpu.SemaphoreType.DMA,
            pltpu.SemaphoreType.DMA,
            pltpu.SemaphoreType.DMA,
            pltpu.SemaphoreType.DMA,
        ],
    )
    def ck(y_hbm, pa_hbm, pb_hbm, wa_hbm, wb_hbm, out_hbm, ia_v, ib_v,
           wa_v, wb_v, a_v, b_v, ga0, ga1, gb0, gb1, oo0, oo1, ww0, ww1):
        wid = lax.axis_index("s") * 2 + lax.axis_index("c")
        base0 = wid * npw
        pltpu.sync_copy(pa_hbm.at[pl.ds(base0, npw)], ia_v)
        pltpu.sync_copy(pb_hbm.at[pl.ds(base0, npw)], ib_v)
        gas = (ga0, ga1)
        gbs = (gb0, gb1)
        oos = (oo0, oo1)
        wws = (ww0, ww1)

        def gather(c):
            sl = pl.ds(c * CCH, CCH)
            hsl = pl.ds(base0 + c * CCH, CCH)
            return (
                pltpu.async_copy(y_hbm.at[ia_v.at[sl]], a_v.at[c % 2],
                                 gas[c % 2]),
                pltpu.async_copy(y_hbm.at[ib_v.at[sl]], b_v.at[c % 2],
                                 gbs[c % 2]),
                pltpu.async_copy(wa_hbm.at[hsl], wa_v.at[c % 2], wws[c % 2]),
                pltpu.async_copy(wb_hbm.at[hsl], wb_v.at[c % 2], wws[c % 2]),
            )

        def put(c):
            return pltpu.async_copy(
                a_v.at[c % 2], out_hbm.at[pl.ds(base0 + c * CCH, CCH)],
                oos[c % 2])

        gd = {0: gather(0)}
        od = {}
        for c in range(nit):
            if c + 1 < nit:
                if c - 1 >= 0:
                    od[c - 1].wait()
                gd[c + 1] = gather(c + 1)
            for dma in gd[c]:
                dma.wait()
            buf = c % 2

            def radd(r, c2, buf=buf):
                wa = wa_v[buf, r, :]
                wb = wb_v[buf, r, :]
                for cc in range(d // 16):
                    sl = pl.ds(cc * 16, 16)
                    a_v[buf, r, sl] = (a_v[buf, r, sl] * wa
                                       + b_v[buf, r, sl] * wb)
                return c2

            lax.fori_loop(0, CCH, radd, 0)
            od[c] = put(c)
        od[nit - 2].wait()
        od[nit - 1].wait()

    return ck


def _dispatch_metadata(route, n):
    e0 = route[:, 0].astype(jnp.int32)
    e1 = route[:, 1].astype(jnp.int32)
    kn = TOPK * n
    nb = kn // T + NUM_E
    ids = jnp.concatenate([e0, e1])
    earange = jnp.arange(NUM_E, dtype=jnp.int32)
    onehot = (ids[:, None] == earange[None, :]).astype(jnp.int32)
    cum = jnp.cumsum(onehot, axis=0)
    rank = jnp.sum(onehot * cum, axis=1) - 1
    counts = cum[-1]
    pc = ((counts + T - 1) // T) * T
    ends = jnp.cumsum(pc)
    starts = ends - pc
    dst = jnp.sum(onehot * starts[None, :], axis=1) + rank
    jT = jnp.arange(nb, dtype=jnp.int32) * T
    be = jnp.sum(jT[:, None] >= ends[None, :], axis=1).astype(jnp.int32)
    be_c = jnp.minimum(be, NUM_E - 1)
    beh = (be_c[:, None] == earange[None, :]).astype(jnp.int32)
    starts_b = jnp.sum(beh * starts[None, :], axis=1)
    counts_b = jnp.sum(beh * counts[None, :], axis=1)
    real = (jT < ends[-1]) & ((jT - starts_b) < counts_b)
    e_last = jnp.max(jnp.where(counts > 0, earange, 0))
    block_e = jnp.where(real, be_c, e_last).astype(jnp.int32)
    flags = real.astype(jnp.int32)
    dst3 = dst.reshape(NW, (kn // NW) // GCH, GCH)
    return dst3, block_e, flags, dst[:n], dst[n:]


def kernel(hidden_states, Wr, br, W1, b1, W2, b2):
    bb, s, h = hidden_states.shape
    n = bb * s
    x2d = hidden_states.reshape(n, h)

    gating, route, aux = _run_router(x2d, Wr, br)
    dst3, block_e, flags, pos_a, pos_b = _dispatch_metadata(route, n)
    nbt = block_e.shape[0] * T

    x_sorted = _make_dispatch(n, nbt, h)(x2d, dst3)
    y_sorted = _run_experts(x_sorted, W1, b1, W2, b2, block_e, flags)
    wa16 = jnp.broadcast_to(route[:, 2:3], (n, 16))
    wb16 = jnp.broadcast_to(route[:, 3:4], (n, 16))
    out2d = _make_combine(n, h)(y_sorted, pos_a, pos_b, wa16, wb16)

    return (out2d.reshape(bb, s, h), aux[0, 0], gating.reshape(bb, s, NUM_E))


# speedup vs baseline: 1.1422x; 1.1422x over previous
import functools

import jax
import jax.numpy as jnp
from jax import lax
from jax.experimental import pallas as pl
from jax.experimental.pallas import tpu as pltpu
from jax.experimental.pallas import tpu_sc as plsc

NUM_E = 8
TOPK = 2
ALPHA = 0.01
T = 512
NW = 32
GCH = 64


def _router_body(x_ref, wr_ref, br_ref, gating_ref, route_ref, aux_ref):
    x = x_ref[...]
    logits = jnp.dot(x, wr_ref[...], preferred_element_type=jnp.float32)
    logits = logits + br_ref[...]
    n = logits.shape[0]
    eidx = lax.broadcasted_iota(jnp.int32, (n, NUM_E), 1)
    m0 = jnp.max(logits, axis=1, keepdims=True)
    e0 = jnp.min(jnp.where(logits == m0, eidx, NUM_E), axis=1, keepdims=True)
    masked = jnp.where(eidx == e0, -jnp.inf, logits)
    m1 = jnp.max(masked, axis=1, keepdims=True)
    e1 = jnp.min(jnp.where(masked == m1, eidx, NUM_E), axis=1, keepdims=True)
    z = jnp.exp(m1 - m0)
    w0 = 1.0 / (1.0 + z)
    w1 = z / (1.0 + z)
    gating = jnp.where(eidx == e0, w0, 0.0) + jnp.where(eidx == e1, w1, 0.0)
    gating_ref[...] = gating
    route_ref[...] = (jnp.where(eidx == 0, e0.astype(jnp.float32), 0.0)
                      + jnp.where(eidx == 1, e1.astype(jnp.float32), 0.0)
                      + jnp.where(eidx == 2, w0, 0.0)
                      + jnp.where(eidx == 3, w1, 0.0))
    imp = jnp.sum(gating, axis=0)
    mean_imp = jnp.mean(imp)
    var = jnp.sum((imp - mean_imp) ** 2) / (NUM_E - 1)
    aux_ref[...] = (ALPHA * var / (mean_imp * mean_imp + 1e-08)).reshape(1, 1)


def _run_router(x2d, Wr, br):
    n = x2d.shape[0]
    return pl.pallas_call(
        _router_body,
        out_shape=[
            jax.ShapeDtypeStruct((n, NUM_E), jnp.float32),
            jax.ShapeDtypeStruct((n, NUM_E), jnp.float32),
            jax.ShapeDtypeStruct((1, 1), jnp.float32),
        ],
    )(x2d, Wr, br.reshape(1, NUM_E))


NSPLIT = 3


def _expert_body(be_ref, fl_ref, x_ref, *rest):
    w1s = rest[:NSPLIT]
    b1s = rest[NSPLIT:2 * NSPLIT]
    w2s = rest[2 * NSPLIT:3 * NSPLIT]
    b2_ref = rest[3 * NSPLIT]
    y_ref = rest[3 * NSPLIT + 1]
    j = pl.program_id(0)

    @pl.when(fl_ref[j] == 1)
    def _():
        x = x_ref[...]
        y = b2_ref[0]
        for k in range(NSPLIT):
            h = jnp.dot(x, w1s[k][0], preferred_element_type=jnp.float32,
                        precision=lax.Precision.DEFAULT)
            h = h + b1s[k][0]
            h = 0.5 * h * (1.0 + lax.erf(h * 0.7071067811865476))
            y = y + jnp.dot(h, w2s[k][0], preferred_element_type=jnp.float32,
                            precision=lax.Precision.DEFAULT)
        y_ref[...] = y


def _run_experts(x_sorted, W1, b1, W2, b2, block_e, flags):
    nbt, h = x_sorted.shape
    nb = nbt // T
    i_dim = W1.shape[2]
    it = i_dim // NSPLIT
    w1_specs = [pl.BlockSpec((1, h, it), lambda j, be, fl, k=k: (be[j], 0, k))
                for k in range(NSPLIT)]
    b1_specs = [pl.BlockSpec((1, 1, it), lambda j, be, fl, k=k: (be[j], 0, k))
                for k in range(NSPLIT)]
    w2_specs = [pl.BlockSpec((1, it, h), lambda j, be, fl, k=k: (be[j], k, 0))
                for k in range(NSPLIT)]
    grid_spec = pltpu.PrefetchScalarGridSpec(
        num_scalar_prefetch=2,
        grid=(nb,),
        in_specs=(
            [pl.BlockSpec((T, h), lambda j, be, fl: (j, 0))]
            + w1_specs + b1_specs + w2_specs
            + [pl.BlockSpec((1, 1, h), lambda j, be, fl: (be[j], 0, 0))]
        ),
        out_specs=pl.BlockSpec((T, h), lambda j, be, fl: (j, 0)),
    )
    b1r = b1.reshape(NUM_E, 1, i_dim)
    return pl.pallas_call(
        _expert_body,
        grid_spec=grid_spec,
        out_shape=jax.ShapeDtypeStruct((nbt, h), jnp.float32),
    )(block_e, flags, x_sorted,
      *([W1] * NSPLIT), *([b1r] * NSPLIT), *([W2] * NSPLIT),
      b2.reshape(NUM_E, 1, h))


def _make_dispatch(n, nbt, d):
    npw = TOPK * n // NW
    nit = npw // GCH
    mesh = plsc.VectorSubcoreMesh(core_axis_name="c", subcore_axis_name="s")

    @functools.partial(
        pl.kernel, mesh=mesh,
        out_type=jax.ShapeDtypeStruct((nbt, d), jnp.float32),
        scratch_types=[
            pltpu.VMEM((nit, GCH), jnp.int32),
            pltpu.VMEM((2, GCH, d), jnp.float32),
            pltpu.SemaphoreType.DMA,
            pltpu.SemaphoreType.DMA,
            pltpu.SemaphoreType.DMA,
            pltpu.SemaphoreType.DMA,
        ],
    )
    def dk(x_hbm, dst3_hbm, out_hbm, idx_v, rows_v, g0, g1, o0, o1):
        wid = lax.axis_index("s") * 2 + lax.axis_index("c")
        src0 = (wid % (NW // TOPK)) * npw
        pltpu.sync_copy(dst3_hbm.at[wid], idx_v)
        gsem = (g0, g1)
        osem = (o0, o1)

        def load(c):
            return pltpu.async_copy(
                x_hbm.at[pl.ds(src0 + c * GCH, GCH)],
                rows_v.at[c % 2], gsem[c % 2])

        def scat(c):
            return pltpu.async_copy(
                rows_v.at[c % 2], out_hbm.at[idx_v.at[c]], osem[c % 2])

        gd = {0: load(0)}
        od = {}
        for c in range(nit):
            if c + 1 < nit:
                if c - 1 >= 0:
                    od[c - 1].wait()
                gd[c + 1] = load(c + 1)
            gd[c].wait()
            od[c] = scat(c)
        od[nit - 2].wait()
        od[nit - 1].wait()

    return dk


CCH = 32


def _make_combine(n, d):
    npw = n // NW
    nit = npw // CCH
    mesh = plsc.VectorSubcoreMesh(core_axis_name="c", subcore_axis_name="s")

    @functools.partial(
        pl.kernel, mesh=mesh,
        out_type=jax.ShapeDtypeStruct((n, d), jnp.float32),
        scratch_types=[
            pltpu.VMEM((npw,), jnp.int32),
            pltpu.VMEM((npw,), jnp.int32),
            pltpu.VMEM((2, CCH, 16), jnp.float32),
            pltpu.VMEM((2, CCH, 16), jnp.float32),
            pltpu.VMEM((2, CCH, d), jnp.float32),
            pltpu.VMEM((2, CCH, d), jnp.float32),
            pltpu.SemaphoreType.DMA,
            pltpu.SemaphoreType.DMA,
            pltpu.SemaphoreType.DMA,
            pltpu.SemaphoreType.DMA,
            pltpu.SemaphoreType.DMA,
            pltpu.SemaphoreType.DMA,
            pltpu.SemaphoreType.DMA,
            pltpu.SemaphoreType.DMA,
        ],
    )
    def ck(y_hbm, pa_hbm, pb_hbm, wa_hbm, wb_hbm, out_hbm, ia_v, ib_v,
           wa_v, wb_v, a_v, b_v, ga0, ga1, gb0, gb1, oo0, oo1, ww0, ww1):
        wid = lax.axis_index("s") * 2 + lax.axis_index("c")
        base0 = wid * npw
        pltpu.sync_copy(pa_hbm.at[pl.ds(base0, npw)], ia_v)
        pltpu.sync_copy(pb_hbm.at[pl.ds(base0, npw)], ib_v)
        gas = (ga0, ga1)
        gbs = (gb0, gb1)
        oos = (oo0, oo1)
        wws = (ww0, ww1)

        def gather(c):
            sl = pl.ds(c * CCH, CCH)
            hsl = pl.ds(base0 + c * CCH, CCH)
            return (
                pltpu.async_copy(y_hbm.at[ia_v.at[sl]], a_v.at[c % 2],
                                 gas[c % 2]),
                pltpu.async_copy(y_hbm.at[ib_v.at[sl]], b_v.at[c % 2],
                                 gbs[c % 2]),
                pltpu.async_copy(wa_hbm.at[hsl], wa_v.at[c % 2], wws[c % 2]),
                pltpu.async_copy(wb_hbm.at[hsl], wb_v.at[c % 2], wws[c % 2]),
            )

        def put(c):
            return pltpu.async_copy(
                a_v.at[c % 2], out_hbm.at[pl.ds(base0 + c * CCH, CCH)],
                oos[c % 2])

        gd = {0: gather(0)}
        od = {}
        for c in range(nit):
            if c + 1 < nit:
                if c - 1 >= 0:
                    od[c - 1].wait()
                gd[c + 1] = gather(c + 1)
            for dma in gd[c]:
                dma.wait()
            buf = c % 2

            def radd(r, c2, buf=buf):
                wa = wa_v[buf, r, :]
                wb = wb_v[buf, r, :]
                for cc in range(d // 16):
                    sl = pl.ds(cc * 16, 16)
                    a_v[buf, r, sl] = (a_v[buf, r, sl] * wa
                                       + b_v[buf, r, sl] * wb)
                return c2

            lax.fori_loop(0, CCH, radd, 0)
            od[c] = put(c)
        od[nit - 2].wait()
        od[nit - 1].wait()

    return ck


def _dispatch_metadata(route, n):
    e0 = route[:, 0].astype(jnp.int32)
    e1 = route[:, 1].astype(jnp.int32)
    kn = TOPK * n
    nb = kn // T + NUM_E
    ids = jnp.concatenate([e0, e1])
    earange = jnp.arange(NUM_E, dtype=jnp.int32)
    onehot = (ids[:, None] == earange[None, :]).astype(jnp.int32)
    cum = jnp.cumsum(onehot, axis=0)
    rank = jnp.sum(onehot * cum, axis=1) - 1
    counts = cum[-1]
    pc = ((counts + T - 1) // T) * T
    ends = jnp.cumsum(pc)
    starts = ends - pc
    dst = jnp.sum(onehot * starts[None, :], axis=1) + rank
    jT = jnp.arange(nb, dtype=jnp.int32) * T
    be = jnp.sum(jT[:, None] >= ends[None, :], axis=1).astype(jnp.int32)
    be_c = jnp.minimum(be, NUM_E - 1)
    beh = (be_c[:, None] == earange[None, :]).astype(jnp.int32)
    starts_b = jnp.sum(beh * starts[None, :], axis=1)
    counts_b = jnp.sum(beh * counts[None, :], axis=1)
    real = (jT < ends[-1]) & ((jT - starts_b) < counts_b)
    e_last = jnp.max(jnp.where(counts > 0, earange, 0))
    block_e = jnp.where(real, be_c, e_last).astype(jnp.int32)
    flags = real.astype(jnp.int32)
    dst3 = dst.reshape(NW, (kn // NW) // GCH, GCH)
    return dst3, block_e, flags, dst[:n], dst[n:]


def kernel(hidden_states, Wr, br, W1, b1, W2, b2):
    bb, s, h = hidden_states.shape
    n = bb * s
    x2d = hidden_states.reshape(n, h)

    gating, route, aux = _run_router(x2d, Wr, br)
    dst3, block_e, flags, pos_a, pos_b = _dispatch_metadata(route, n)
    nbt = block_e.shape[0] * T

    x_sorted = _make_dispatch(n, nbt, h)(x2d, dst3)
    y_sorted = _run_experts(x_sorted, W1, b1, W2, b2, block_e, flags)
    wa16 = jnp.broadcast_to(route[:, 2:3], (n, 16))
    wb16 = jnp.broadcast_to(route[:, 3:4], (n, 16))
    out2d = _make_combine(n, h)(y_sorted, pos_a, pos_b, wa16, wb16)

    return (out2d.reshape(bb, s, h), aux[0, 0], gating.reshape(bb, s, NUM_E))


# speedup vs baseline: 1.1726x; 1.0266x over previous
import functools

import jax
import jax.numpy as jnp
from jax import lax
from jax.experimental import pallas as pl
from jax.experimental.pallas import tpu as pltpu
from jax.experimental.pallas import tpu_sc as plsc

NUM_E = 8
TOPK = 2
ALPHA = 0.01
T = 512
NW = 32
GCH = 64


def _router_body(x_ref, wr_ref, br_ref, gating_ref, route_ref, aux_ref):
    x = x_ref[...]
    logits = jnp.dot(x, wr_ref[...], preferred_element_type=jnp.float32)
    logits = logits + br_ref[...]
    n = logits.shape[0]
    eidx = lax.broadcasted_iota(jnp.int32, (n, NUM_E), 1)
    m0 = jnp.max(logits, axis=1, keepdims=True)
    e0 = jnp.min(jnp.where(logits == m0, eidx, NUM_E), axis=1, keepdims=True)
    masked = jnp.where(eidx == e0, -jnp.inf, logits)
    m1 = jnp.max(masked, axis=1, keepdims=True)
    e1 = jnp.min(jnp.where(masked == m1, eidx, NUM_E), axis=1, keepdims=True)
    z = jnp.exp(m1 - m0)
    w0 = 1.0 / (1.0 + z)
    w1 = z / (1.0 + z)
    gating = jnp.where(eidx == e0, w0, 0.0) + jnp.where(eidx == e1, w1, 0.0)
    gating_ref[...] = gating
    route_ref[...] = (jnp.where(eidx == 0, e0.astype(jnp.float32), 0.0)
                      + jnp.where(eidx == 1, e1.astype(jnp.float32), 0.0)
                      + jnp.where(eidx == 2, w0, 0.0)
                      + jnp.where(eidx == 3, w1, 0.0))
    imp = jnp.sum(gating, axis=0)
    mean_imp = jnp.mean(imp)
    var = jnp.sum((imp - mean_imp) ** 2) / (NUM_E - 1)
    aux_ref[...] = (ALPHA * var / (mean_imp * mean_imp + 1e-08)).reshape(1, 1)


def _run_router(x2d, Wr, br):
    n = x2d.shape[0]
    return pl.pallas_call(
        _router_body,
        out_shape=[
            jax.ShapeDtypeStruct((n, NUM_E), jnp.float32),
            jax.ShapeDtypeStruct((n, NUM_E), jnp.float32),
            jax.ShapeDtypeStruct((1, 1), jnp.float32),
        ],
    )(x2d, Wr, br.reshape(1, NUM_E))


NSPLIT = 1


def _expert_body(be_ref, fl_ref, x_ref, *rest):
    w1s = rest[:NSPLIT]
    b1s = rest[NSPLIT:2 * NSPLIT]
    w2s = rest[2 * NSPLIT:3 * NSPLIT]
    b2_ref = rest[3 * NSPLIT]
    y_ref = rest[3 * NSPLIT + 1]
    j = pl.program_id(0)

    @pl.when(fl_ref[j] == 1)
    def _():
        x = x_ref[...]
        y = b2_ref[0]
        for k in range(NSPLIT):
            h = jnp.dot(x, w1s[k][0], preferred_element_type=jnp.float32,
                        precision=lax.Precision.DEFAULT)
            h = h + b1s[k][0]
            h = 0.5 * h * (1.0 + lax.erf(h * 0.7071067811865476))
            y = y + jnp.dot(h, w2s[k][0], preferred_element_type=jnp.float32,
                            precision=lax.Precision.DEFAULT)
        y_ref[...] = y


def _run_experts(x_sorted, W1, b1, W2, b2, block_e, flags):
    nbt, h = x_sorted.shape
    nb = nbt // T
    i_dim = W1.shape[2]
    it = i_dim // NSPLIT
    w1_specs = [pl.BlockSpec((1, h, it), lambda j, be, fl, k=k: (be[j], 0, k))
                for k in range(NSPLIT)]
    b1_specs = [pl.BlockSpec((1, 1, it), lambda j, be, fl, k=k: (be[j], 0, k))
                for k in range(NSPLIT)]
    w2_specs = [pl.BlockSpec((1, it, h), lambda j, be, fl, k=k: (be[j], k, 0))
                for k in range(NSPLIT)]
    grid_spec = pltpu.PrefetchScalarGridSpec(
        num_scalar_prefetch=2,
        grid=(nb,),
        in_specs=(
            [pl.BlockSpec((T, h), lambda j, be, fl: (j, 0))]
            + w1_specs + b1_specs + w2_specs
            + [pl.BlockSpec((1, 1, h), lambda j, be, fl: (be[j], 0, 0))]
        ),
        out_specs=pl.BlockSpec((T, h), lambda j, be, fl: (j, 0)),
    )
    b1r = b1.reshape(NUM_E, 1, i_dim)
    return pl.pallas_call(
        _expert_body,
        grid_spec=grid_spec,
        out_shape=jax.ShapeDtypeStruct((nbt, h), jnp.float32),
    )(block_e, flags, x_sorted,
      *([W1] * NSPLIT), *([b1r] * NSPLIT), *([W2] * NSPLIT),
      b2.reshape(NUM_E, 1, h))


def _make_dispatch(n, nbt, d):
    npw = TOPK * n // NW
    nit = npw // GCH
    mesh = plsc.VectorSubcoreMesh(core_axis_name="c", subcore_axis_name="s")

    @functools.partial(
        pl.kernel, mesh=mesh,
        out_type=jax.ShapeDtypeStruct((nbt, d), jnp.float32),
        scratch_types=[
            pltpu.VMEM((nit, GCH), jnp.int32),
            pltpu.VMEM((2, GCH, d), jnp.float32),
            pltpu.SemaphoreType.DMA,
            pltpu.SemaphoreType.DMA,
            pltpu.SemaphoreType.DMA,
            pltpu.SemaphoreType.DMA,
        ],
    )
    def dk(x_hbm, dst3_hbm, out_hbm, idx_v, rows_v, g0, g1, o0, o1):
        wid = lax.axis_index("s") * 2 + lax.axis_index("c")
        src0 = (wid % (NW // TOPK)) * npw
        pltpu.sync_copy(dst3_hbm.at[wid], idx_v)
        gsem = (g0, g1)
        osem = (o0, o1)

        def load(c):
            return pltpu.async_copy(
                x_hbm.at[pl.ds(src0 + c * GCH, GCH)],
                rows_v.at[c % 2], gsem[c % 2])

        def scat(c):
            return pltpu.async_copy(
                rows_v.at[c % 2], out_hbm.at[idx_v.at[c]], osem[c % 2])

        gd = {0: load(0)}
        od = {}
        for c in range(nit):
            if c + 1 < nit:
                if c - 1 >= 0:
                    od[c - 1].wait()
                gd[c + 1] = load(c + 1)
            gd[c].wait()
            od[c] = scat(c)
        od[nit - 2].wait()
        od[nit - 1].wait()

    return dk


CCH = 32


def _make_combine(n, d):
    npw = n // NW
    nit = npw // CCH
    mesh = plsc.VectorSubcoreMesh(core_axis_name="c", subcore_axis_name="s")

    @functools.partial(
        pl.kernel, mesh=mesh,
        out_type=jax.ShapeDtypeStruct((n, d), jnp.float32),
        scratch_types=[
            pltpu.VMEM((npw,), jnp.int32),
            pltpu.VMEM((npw,), jnp.int32),
            pltpu.VMEM((2, CCH, 16), jnp.float32),
            pltpu.VMEM((2, CCH, 16), jnp.float32),
            pltpu.VMEM((2, CCH, d), jnp.float32),
            pltpu.VMEM((2, CCH, d), jnp.float32),
            pltpu.SemaphoreType.DMA,
            pltpu.SemaphoreType.DMA,
            pltpu.SemaphoreType.DMA,
            pltpu.SemaphoreType.DMA,
            pltpu.SemaphoreType.DMA,
            pltpu.SemaphoreType.DMA,
            pltpu.SemaphoreType.DMA,
            pltpu.SemaphoreType.DMA,
        ],
    )
    def ck(y_hbm, pa_hbm, pb_hbm, wa_hbm, wb_hbm, out_hbm, ia_v, ib_v,
           wa_v, wb_v, a_v, b_v, ga0, ga1, gb0, gb1, oo0, oo1, ww0, ww1):
        wid = lax.axis_index("s") * 2 + lax.axis_index("c")
        base0 = wid * npw
        pltpu.sync_copy(pa_hbm.at[pl.ds(base0, npw)], ia_v)
        pltpu.sync_copy(pb_hbm.at[pl.ds(base0, npw)], ib_v)
        gas = (ga0, ga1)
        gbs = (gb0, gb1)
        oos = (oo0, oo1)
        wws = (ww0, ww1)

        def gather(c):
            sl = pl.ds(c * CCH, CCH)
            hsl = pl.ds(base0 + c * CCH, CCH)
            return (
                pltpu.async_copy(y_hbm.at[ia_v.at[sl]], a_v.at[c % 2],
                                 gas[c % 2]),
                pltpu.async_copy(y_hbm.at[ib_v.at[sl]], b_v.at[c % 2],
                                 gbs[c % 2]),
                pltpu.async_copy(wa_hbm.at[hsl], wa_v.at[c % 2], wws[c % 2]),
                pltpu.async_copy(wb_hbm.at[hsl], wb_v.at[c % 2], wws[c % 2]),
            )

        def put(c):
            return pltpu.async_copy(
                a_v.at[c % 2], out_hbm.at[pl.ds(base0 + c * CCH, CCH)],
                oos[c % 2])

        gd = {0: gather(0)}
        od = {}
        for c in range(nit):
            if c + 1 < nit:
                if c - 1 >= 0:
                    od[c - 1].wait()
                gd[c + 1] = gather(c + 1)
            for dma in gd[c]:
                dma.wait()
            buf = c % 2

            def radd(r, c2, buf=buf):
                wa = wa_v[buf, r, :]
                wb = wb_v[buf, r, :]
                for cc in range(d // 16):
                    sl = pl.ds(cc * 16, 16)
                    a_v[buf, r, sl] = (a_v[buf, r, sl] * wa
                                       + b_v[buf, r, sl] * wb)
                return c2

            lax.fori_loop(0, CCH, radd, 0)
            od[c] = put(c)
        od[nit - 2].wait()
        od[nit - 1].wait()

    return ck


def _dispatch_metadata(route, n):
    e0 = route[:, 0].astype(jnp.int32)
    e1 = route[:, 1].astype(jnp.int32)
    kn = TOPK * n
    nb = kn // T + NUM_E
    ids = jnp.concatenate([e0, e1])
    earange = jnp.arange(NUM_E, dtype=jnp.int32)
    onehot = (ids[:, None] == earange[None, :]).astype(jnp.int32)
    cum = jnp.cumsum(onehot, axis=0)
    rank = jnp.sum(onehot * cum, axis=1) - 1
    counts = cum[-1]
    pc = ((counts + T - 1) // T) * T
    ends = jnp.cumsum(pc)
    starts = ends - pc
    dst = jnp.sum(onehot * starts[None, :], axis=1) + rank
    jT = jnp.arange(nb, dtype=jnp.int32) * T
    be = jnp.sum(jT[:, None] >= ends[None, :], axis=1).astype(jnp.int32)
    be_c = jnp.minimum(be, NUM_E - 1)
    beh = (be_c[:, None] == earange[None, :]).astype(jnp.int32)
    starts_b = jnp.sum(beh * starts[None, :], axis=1)
    counts_b = jnp.sum(beh * counts[None, :], axis=1)
    real = (jT < ends[-1]) & ((jT - starts_b) < counts_b)
    e_last = jnp.max(jnp.where(counts > 0, earange, 0))
    block_e = jnp.where(real, be_c, e_last).astype(jnp.int32)
    flags = real.astype(jnp.int32)
    dst3 = dst.reshape(NW, (kn // NW) // GCH, GCH)
    return dst3, block_e, flags, dst[:n], dst[n:]


def kernel(hidden_states, Wr, br, W1, b1, W2, b2):
    bb, s, h = hidden_states.shape
    n = bb * s
    x2d = hidden_states.reshape(n, h)

    gating, route, aux = _run_router(x2d, Wr, br)
    dst3, block_e, flags, pos_a, pos_b = _dispatch_metadata(route, n)
    nbt = block_e.shape[0] * T

    x_sorted = _make_dispatch(n, nbt, h)(x2d, dst3)
    y_sorted = _run_experts(x_sorted, W1, b1, W2, b2, block_e, flags)
    wa16 = jnp.broadcast_to(route[:, 2:3], (n, 16))
    wb16 = jnp.broadcast_to(route[:, 3:4], (n, 16))
    out2d = _make_combine(n, h)(y_sorted, pos_a, pos_b, wa16, wb16)

    return (out2d.reshape(bb, s, h), aux[0, 0], gating.reshape(bb, s, NUM_E))
